# Initial kernel scaffold; baseline (speedup 1.0000x reference)
import jax
import jax.numpy as jnp
from jax.experimental import pallas as pl


def kernel(images, boxes, classes):
    raise NotImplementedError("write your pallas kernel here")


# speedup vs baseline: 14.6905x; 14.6905x over previous
import functools

import numpy as np
import jax
import jax.numpy as jnp
from jax import lax
from jax.experimental import pallas as pl
from jax.experimental.pallas import tpu as pltpu
from jax.experimental.pallas import tpu_sc as plsc

MIN_LEVEL = 3
MAX_LEVEL = 7
NUM_SCALES = 3
ASPECT_RATIOS = (0.5, 1.0, 2.0)
ANCHOR_SIZE = 4.0
POS_T = 0.5
NEG_T = 0.4
BOX_VARIANCE = np.array([0.1, 0.1, 0.2, 0.2], dtype=np.float32)
IMG_H = 512
IMG_W = 512

NUM_CORES = 2
NUM_SUBCORES = 16
LANES = 16
NW = NUM_CORES * NUM_SUBCORES


def _gen_anchors_np(image_h, image_w):
    all_anchors = []
    for level in range(MIN_LEVEL, MAX_LEVEL + 1):
        stride = 2 ** level
        fh = image_h // stride
        fw = image_w // stride
        cx = (np.arange(fw, dtype=np.float32) + 0.5) * stride
        cy = (np.arange(fh, dtype=np.float32) + 0.5) * stride
        cxg, cyg = np.meshgrid(cx, cy)
        dims = []
        for s in range(NUM_SCALES):
            size = ANCHOR_SIZE * stride * (2.0 ** (float(s) / NUM_SCALES))
            for ar in ASPECT_RATIOS:
                dims.append((size * np.sqrt(ar), size / np.sqrt(ar)))
        dims = np.array(dims, dtype=np.float32)
        centers = np.stack([cxg, cyg], axis=-1).reshape(-1, 1, 2)
        wh = dims.reshape(1, -1, 2)
        anchors = np.concatenate(
            [centers - wh / 2.0, centers + wh / 2.0], axis=-1).reshape(-1, 4)
        all_anchors.append(anchors)
    return np.concatenate(all_anchors, axis=0).astype(np.float32)


_ANCHORS = _gen_anchors_np(IMG_H, IMG_W)
M = _ANCHORS.shape[0]
M_PAD = ((M + 16 * NW - 1) // (16 * NW)) * (16 * NW)
CHUNK = M_PAD // NW
VPC = CHUNK // LANES
_PAD_ROWS = np.tile(_ANCHORS[:1], (M_PAD - M, 1))
_ANCH_PAD = np.concatenate([_ANCHORS, _PAD_ROWS], axis=0)

_A_X1 = _ANCH_PAD[:, 0]
_A_Y1 = _ANCH_PAD[:, 1]
_A_X2 = _ANCH_PAD[:, 2]
_A_Y2 = _ANCH_PAD[:, 3]
_A_AREA = (_A_X2 - _A_X1) * (_A_Y2 - _A_Y1)
_ANCH_SC = np.stack([_A_X1, _A_Y1, _A_X2, _A_Y2, _A_AREA], axis=0)
_A_W = _A_X2 - _A_X1
_A_H = _A_Y2 - _A_Y1
_A_CX = _A_X1 + 0.5 * _A_W
_A_CY = _A_Y1 + 0.5 * _A_H
_ANCH_TC = np.stack([_A_CY, _A_CX, _A_H, _A_W], axis=0).reshape(4, M_PAD // 128, 128)


def _sc_match(B, N):
    mesh = plsc.VectorSubcoreMesh(core_axis_name="c", subcore_axis_name="s")
    L = LANES

    def body(gt_hbm, anch_hbm, out_hbm, gt_v, area_v, anch_v, out_v):
        wid = lax.axis_index("s") * NUM_CORES + lax.axis_index("c")
        base_m = wid * CHUNK
        for q in range(5):
            pltpu.sync_copy(anch_hbm.at[pl.ds(q * M_PAD + base_m, CHUNK)],
                            anch_v.at[pl.ds(q * CHUNK, CHUNK)])
        pltpu.sync_copy(gt_hbm, gt_v)

        def area_body(n, _):
            for b in range(B):
                base = (b * 5 * N + n) * L
                gx1 = gt_v[pl.ds(base, L)]
                gy1 = gt_v[pl.ds(base + N * L, L)]
                gx2 = gt_v[pl.ds(base + 2 * N * L, L)]
                gy2 = gt_v[pl.ds(base + 3 * N * L, L)]
                area_v[pl.ds((b * N + n) * L, L)] = (gx2 - gx1) * (gy2 - gy1)
            return 0
        lax.fori_loop(0, N, area_body, 0)

        lane = lax.iota(jnp.int32, L)
        for b in range(B):
            def v_body(v, _, b=b):
                mb = v * L
                ax1 = anch_v[pl.ds(mb, L)]
                ay1 = anch_v[pl.ds(CHUNK + mb, L)]
                ax2 = anch_v[pl.ds(2 * CHUNK + mb, L)]
                ay2 = anch_v[pl.ds(3 * CHUNK + mb, L)]
                aarea = anch_v[pl.ds(4 * CHUNK + mb, L)]

                def n_body(n, st):
                    biou, bidx = st
                    gbase = (b * 5 * N + n) * L
                    gx1 = gt_v[pl.ds(gbase, L)]
                    gy1 = gt_v[pl.ds(gbase + N * L, L)]
                    gx2 = gt_v[pl.ds(gbase + 2 * N * L, L)]
                    gy2 = gt_v[pl.ds(gbase + 3 * N * L, L)]
                    garea = area_v[pl.ds((b * N + n) * L, L)]
                    w = jnp.maximum(
                        jnp.minimum(ax2, gx2) - jnp.maximum(ax1, gx1), 0.0)
                    h = jnp.maximum(
                        jnp.minimum(ay2, gy2) - jnp.maximum(ay1, gy1), 0.0)
                    inter = w * h
                    union = aarea + garea - inter
                    iou = inter / jnp.maximum(union, 1e-8)
                    upd = iou > biou
                    return (jnp.where(upd, iou, biou),
                            jnp.where(upd, n, bidx))

                biou, bidx = lax.fori_loop(
                    0, N, n_body,
                    (jnp.full((L,), -1.0, jnp.float32),
                     jnp.zeros((L,), jnp.int32)))
                flat = bidx * L + lane
                for q in range(5):
                    out_v[pl.ds((q * B + b) * CHUNK + mb, L)] = (
                        plsc.load_gather(gt_v, [flat + (b * 5 + q) * N * L]))
                out_v[pl.ds((5 * B + b) * CHUNK + mb, L)] = biou
                return 0
            lax.fori_loop(0, VPC, v_body, 0)

        for q in range(6):
            for b in range(B):
                pltpu.sync_copy(
                    out_v.at[pl.ds((q * B + b) * CHUNK, CHUNK)],
                    out_hbm.at[pl.ds((q * B + b) * M_PAD + base_m, CHUNK)])

    return pl.kernel(
        body,
        out_type=jax.ShapeDtypeStruct((6 * B * M_PAD,), jnp.float32),
        mesh=mesh,
        compiler_params=pltpu.CompilerParams(needs_layout_passes=False),
        scratch_types=[
            pltpu.VMEM((B * 5 * N * L,), jnp.float32),
            pltpu.VMEM((B * N * L,), jnp.float32),
            pltpu.VMEM((5 * CHUNK,), jnp.float32),
            pltpu.VMEM((6 * B * CHUNK,), jnp.float32),
        ],
    )


def _tc_encode_body(B, m_ref, a_ref, bt_ref, ct_ref):
    acy = a_ref[0]
    acx = a_ref[1]
    ah = a_ref[2]
    aw = a_ref[3]
    for b in range(B):
        mx1 = m_ref[0, b]
        my1 = m_ref[1, b]
        mx2 = m_ref[2, b]
        my2 = m_ref[3, b]
        mcls = m_ref[4, b]
        miou = m_ref[5, b]
        gw = mx2 - mx1
        gh = my2 - my1
        gcx = mx1 + 0.5 * gw
        gcy = my1 + 0.5 * gh
        ty = ((gcy - acy) / ah) / BOX_VARIANCE[0]
        tx = ((gcx - acx) / aw) / BOX_VARIANCE[1]
        th = jnp.log(gh / ah) / BOX_VARIANCE[2]
        tw = jnp.log(gw / aw) / BOX_VARIANCE[3]
        pos = miou >= POS_T
        neg = miou < NEG_T
        bt_ref[b, 0] = jnp.where(pos, ty, 0.0)
        bt_ref[b, 1] = jnp.where(pos, tx, 0.0)
        bt_ref[b, 2] = jnp.where(pos, th, 0.0)
        bt_ref[b, 3] = jnp.where(pos, tw, 0.0)
        ct_ref[b] = jnp.where(pos, mcls, jnp.where(neg, -1.0, -2.0))


def kernel(images, boxes, classes):
    B, N = boxes.shape[0], boxes.shape[1]
    del images

    gt = jnp.stack(
        [boxes[..., 0], boxes[..., 1], boxes[..., 2], boxes[..., 3],
         classes.astype(jnp.float32)], axis=1)
    gtb = jnp.broadcast_to(gt[..., None], (B, 5, N, LANES)).reshape(-1)

    matched = _sc_match(B, N)(gtb, jnp.asarray(_ANCH_SC.reshape(-1)))

    rows = M_PAD // 128
    m4 = matched.reshape(6, B, rows, 128)
    bt, ct = pl.pallas_call(
        functools.partial(_tc_encode_body, B),
        out_shape=[
            jax.ShapeDtypeStruct((B, 4, rows, 128), jnp.float32),
            jax.ShapeDtypeStruct((B, rows, 128), jnp.float32),
        ],
    )(m4, jnp.asarray(_ANCH_TC))

    box_targets = bt.reshape(B, 4, M_PAD)[:, :, :M].transpose(0, 2, 1)
    class_targets = ct.reshape(B, M_PAD)[:, :M, None]
    return box_targets, class_targets
